# baseline (speedup 1.0000x reference)
import jax
import jax.numpy as jnp
from jax.experimental import pallas as pl
from jax.experimental.pallas import tpu as pltpu

_B = 4096
_BT = 1024
_NB = _B // _BT
_G = 4
_CC = 1024 // _G


def _mlp_kernel(num_ref, emb8_ref, w0c_ref, w0t_ref, b0_ref,
                w1_ref, b1_ref, w2_ref, b2_ref, w3_ref, b3_ref,
                out_ref, c0_ref, w1b_ref, w2b_ref, w3b_ref):
    bf = jnp.bfloat16
    n_num = num_ref.shape[1]
    g = pl.program_id(0)

    for k in range(_G):
        @pl.when(g == k)
        def _():
            chunk = jnp.dot(emb8_ref[...], w0c_ref[...],
                            preferred_element_type=jnp.float32)
            c0_ref[:, k * _CC:(k + 1) * _CC] = (
                chunk + b0_ref[:, k * _CC:(k + 1) * _CC])

    @pl.when(g == 0)
    def _():
        w1b_ref[...] = w1_ref[...].astype(bf)
        w2b_ref[...] = w2_ref[...].astype(bf)
        w3b_ref[...] = w3_ref[...].astype(bf)

    @pl.when(g >= _G)
    def _():
        x = num_ref[...].astype(bf)
        w0t = w0t_ref[0:n_num, :].astype(bf)
        h = jnp.dot(x, w0t, preferred_element_type=jnp.float32)
        h = jnp.maximum(h + c0_ref[0:1, :], 0.0).astype(bf)
        h = jnp.maximum(jnp.dot(h, w1b_ref[...],
                                preferred_element_type=jnp.float32)
                        + b1_ref[...], 0.0).astype(bf)
        h = jnp.maximum(jnp.dot(h, w2b_ref[...],
                                preferred_element_type=jnp.float32)
                        + b2_ref[...], 0.0).astype(bf)
        out_ref[...] = jnp.dot(h, w3b_ref[...],
                               preferred_element_type=jnp.float32) + b3_ref[...]


def kernel(numerical_features, categorical_features, emb_tables,
           W0, b0, W1, b1, W2, b2, W3, b3):
    del categorical_features
    n_num = numerical_features.shape[1]
    d_in = W0.shape[0]
    emb_flat = emb_tables[:, 0, :].reshape(1, -1)
    emb_pad = jnp.pad(emb_flat, ((0, 0), (n_num, 0)))
    emb8 = jnp.broadcast_to(emb_pad, (8, d_in))

    const = lambda i: (0, 0)
    tile = lambda i: (jnp.maximum(i - _G, 0), 0)
    out = pl.pallas_call(
        _mlp_kernel,
        grid=(_G + _NB,),
        in_specs=[
            pl.BlockSpec((_BT, n_num), tile),
            pl.BlockSpec(emb8.shape, const),
            pl.BlockSpec((d_in, _CC), lambda i: (0, jnp.minimum(i, _G - 1))),
            pl.BlockSpec((16, 1024), const),
            pl.BlockSpec((1, b0.shape[0]), const),
            pl.BlockSpec(W1.shape, const),
            pl.BlockSpec((1, b1.shape[0]), const),
            pl.BlockSpec(W2.shape, const),
            pl.BlockSpec((1, b2.shape[0]), const),
            pl.BlockSpec(W3.shape, const),
            pl.BlockSpec((1, 1), const),
        ],
        out_specs=pl.BlockSpec((_BT, 1), tile),
        out_shape=jax.ShapeDtypeStruct((_B, 1), jnp.float32),
        scratch_shapes=[
            pltpu.VMEM((8, b0.shape[0]), jnp.float32),
            pltpu.VMEM(W1.shape, jnp.bfloat16),
            pltpu.VMEM(W2.shape, jnp.bfloat16),
            pltpu.VMEM(W3.shape, jnp.bfloat16),
        ],
    )(numerical_features, emb8, W0, W0, b0.reshape(1, -1),
      W1, b1.reshape(1, -1), W2, b2.reshape(1, -1), W3, b3.reshape(1, -1))
    return out[:, 0]


# speedup vs baseline: 1.0795x; 1.0795x over previous
import jax
import jax.numpy as jnp
from jax.experimental import pallas as pl
from jax.experimental.pallas import tpu as pltpu

_B = 4096
_BT = 2048
_NB = _B // _BT


def _mlp_kernel(num_ref, emb_ref, w0_ref, b0_ref,
                w1_ref, b1_ref, w2_ref, b2_ref, w3_ref, b3_ref,
                out_ref, c0_ref, w1b_ref, w2b_ref, w3b_ref):
    bf = jnp.bfloat16
    n_num = num_ref.shape[1]
    n_tab = emb_ref.shape[0]

    @pl.when(pl.program_id(0) == 0)
    def _():
        parts = [jnp.zeros((1, n_num), jnp.float32)]
        parts += [emb_ref[i, :, :] for i in range(n_tab)]
        emb_pad = jnp.concatenate(parts, axis=1)
        c0 = jnp.dot(emb_pad, w0_ref[...], preferred_element_type=jnp.float32)
        c0_ref[...] = c0 + b0_ref[...]
        w1b_ref[...] = w1_ref[...].astype(bf)
        w2b_ref[...] = w2_ref[...].astype(bf)
        w3b_ref[...] = w3_ref[...].astype(bf)

    x = num_ref[...].astype(bf)
    w0t = w0_ref[0:n_num, :].astype(bf)
    h = jnp.dot(x, w0t, preferred_element_type=jnp.float32)
    h = jnp.maximum(h + c0_ref[0:1, :], 0.0).astype(bf)
    h = jnp.maximum(jnp.dot(h, w1b_ref[...],
                            preferred_element_type=jnp.float32)
                    + b1_ref[...], 0.0).astype(bf)
    h = jnp.maximum(jnp.dot(h, w2b_ref[...],
                            preferred_element_type=jnp.float32)
                    + b2_ref[...], 0.0).astype(bf)
    out_ref[...] = jnp.dot(h, w3b_ref[...],
                           preferred_element_type=jnp.float32) + b3_ref[...]


def kernel(numerical_features, categorical_features, emb_tables,
           W0, b0, W1, b1, W2, b2, W3, b3):
    del categorical_features
    n_num = numerical_features.shape[1]

    const = lambda i: (0, 0)
    const3 = lambda i: (0, 0, 0)
    out = pl.pallas_call(
        _mlp_kernel,
        grid=(_NB,),
        in_specs=[
            pl.BlockSpec((_BT, n_num), lambda i: (i, 0)),
            pl.BlockSpec(emb_tables.shape, const3),
            pl.BlockSpec(W0.shape, const),
            pl.BlockSpec((1, b0.shape[0]), const),
            pl.BlockSpec(W1.shape, const),
            pl.BlockSpec((1, b1.shape[0]), const),
            pl.BlockSpec(W2.shape, const),
            pl.BlockSpec((1, b2.shape[0]), const),
            pl.BlockSpec(W3.shape, const),
            pl.BlockSpec((1, 1), const),
        ],
        out_specs=pl.BlockSpec((_BT, 1), lambda i: (i, 0)),
        out_shape=jax.ShapeDtypeStruct((_B, 1), jnp.float32),
        scratch_shapes=[
            pltpu.VMEM((1, b0.shape[0]), jnp.float32),
            pltpu.VMEM(W1.shape, jnp.bfloat16),
            pltpu.VMEM(W2.shape, jnp.bfloat16),
            pltpu.VMEM(W3.shape, jnp.bfloat16),
        ],
    )(numerical_features, emb_tables, W0, b0.reshape(1, -1),
      W1, b1.reshape(1, -1), W2, b2.reshape(1, -1), W3, b3.reshape(1, -1))
    return out[:, 0]


# speedup vs baseline: 1.4607x; 1.3532x over previous
import jax
import jax.numpy as jnp
from jax import lax
from jax.experimental import pallas as pl
from jax.experimental.pallas import tpu as pltpu

_B = 4096
_BT = 2048
_NB = _B // _BT


def _mlp_kernel(numt_ref, emb_ref, w0_ref, b0_ref,
                w1_ref, b1_ref, w2_ref, b2_ref, w3t_ref, b3_ref,
                out_ref, c0_ref, w1b_ref, w2b_ref, w3b_ref):
    bf = jnp.bfloat16
    n_num = numt_ref.shape[0]
    n_tab = emb_ref.shape[0]

    @pl.when(pl.program_id(0) == 0)
    def _():
        parts = [jnp.zeros((1, n_num), jnp.float32)]
        parts += [emb_ref[i, :, :] for i in range(n_tab)]
        emb_pad = jnp.concatenate(parts, axis=1)
        c0 = jnp.dot(emb_pad, w0_ref[...], preferred_element_type=jnp.float32)
        c0_ref[...] = c0 + b0_ref[...]
        w1b_ref[...] = w1_ref[...].astype(bf)
        w2b_ref[...] = w2_ref[...].astype(bf)
        w3b_ref[...] = w3t_ref[...].T.astype(bf)

    xt = numt_ref[...].astype(bf)
    w0t = w0_ref[0:n_num, :].astype(bf)
    h = lax.dot_general(xt, w0t, (((0,), (0,)), ((), ())),
                        preferred_element_type=jnp.float32)
    h = jnp.maximum(h + c0_ref[0:1, :], 0.0).astype(bf)
    h = jnp.maximum(jnp.dot(h, w1b_ref[...],
                            preferred_element_type=jnp.float32)
                    + b1_ref[...], 0.0).astype(bf)
    h = jnp.maximum(jnp.dot(h, w2b_ref[...],
                            preferred_element_type=jnp.float32)
                    + b2_ref[...], 0.0).astype(bf)
    o = jnp.dot(h, w3b_ref[...], preferred_element_type=jnp.float32)
    out_ref[...] = (o + b3_ref[...])[:, 0]


def kernel(numerical_features, categorical_features, emb_tables,
           W0, b0, W1, b1, W2, b2, W3, b3):
    del categorical_features
    n_num = numerical_features.shape[1]

    const = lambda i: (0, 0)
    const3 = lambda i: (0, 0, 0)
    out = pl.pallas_call(
        _mlp_kernel,
        grid=(_NB,),
        in_specs=[
            pl.BlockSpec((n_num, _BT), lambda i: (0, i)),
            pl.BlockSpec(emb_tables.shape, const3),
            pl.BlockSpec(W0.shape, const),
            pl.BlockSpec((1, b0.shape[0]), const),
            pl.BlockSpec(W1.shape, const),
            pl.BlockSpec((1, b1.shape[0]), const),
            pl.BlockSpec(W2.shape, const),
            pl.BlockSpec((1, b2.shape[0]), const),
            pl.BlockSpec((1, W3.shape[0]), const),
            pl.BlockSpec((1, 1), const),
        ],
        out_specs=pl.BlockSpec((_BT,), lambda i: (i,)),
        out_shape=jax.ShapeDtypeStruct((_B,), jnp.float32),
        scratch_shapes=[
            pltpu.VMEM((1, b0.shape[0]), jnp.float32),
            pltpu.VMEM(W1.shape, jnp.bfloat16),
            pltpu.VMEM(W2.shape, jnp.bfloat16),
            pltpu.VMEM((W3.shape[0], 1), jnp.bfloat16),
        ],
    )(numerical_features.T, emb_tables, W0, b0.reshape(1, -1),
      W1, b1.reshape(1, -1), W2, b2.reshape(1, -1), W3.T, b3.reshape(1, 1))
    return out


# speedup vs baseline: 1.5504x; 1.0614x over previous
import jax
import jax.numpy as jnp
from jax import lax
from jax.experimental import pallas as pl
from jax.experimental.pallas import tpu as pltpu

_B = 4096
_BT = 2048
_NB = _B // _BT
_W0_ROWS = 3341
_CHUNKS = [(0, 840), (840, 840), (1680, 840), (2520, 821)]


def _w0_copies(w0_any, w0_v, sem):
    return [pltpu.make_async_copy(w0_any.at[pl.ds(s, n), :],
                                  w0_v.at[pl.ds(s, n), :], sem)
            for s, n in _CHUNKS]


def _mlp_kernel(numt_ref, emb_ref, w0t16_ref, b0_ref,
                w1_ref, b1_ref, w2_ref, b2_ref, w3t_ref, b3_ref,
                w0_any,
                out_ref,
                p_ref, c0_ref, w1b_ref, w2b_ref, w3b_ref, w0_v, sem):
    bf = jnp.bfloat16
    n_num = numt_ref.shape[0]
    n_tab = emb_ref.shape[0]
    g = pl.program_id(0)

    @pl.when(g == 0)
    def _():
        for c in _w0_copies(w0_any, w0_v, sem):
            c.start()

    @pl.when(g < _NB)
    def _():
        xt = numt_ref[...].astype(bf)
        w0t = w0t16_ref[0:n_num, :].astype(bf)
        p = lax.dot_general(xt, w0t, (((0,), (0,)), ((), ())),
                            preferred_element_type=jnp.float32)
        p_ref[pl.ds(g * _BT, _BT), :] = p.astype(bf)

    @pl.when(g == _NB)
    def _():
        for c in _w0_copies(w0_any, w0_v, sem):
            c.wait()
        parts = [jnp.zeros((1, n_num), jnp.float32)]
        parts += [emb_ref[i, :, :] for i in range(n_tab)]
        emb_pad = jnp.concatenate(parts, axis=1)
        c0 = jnp.dot(emb_pad, w0_v[...], preferred_element_type=jnp.float32)
        c0_ref[...] = (c0 + b0_ref[...]).astype(bf)
        w1b_ref[...] = w1_ref[...].astype(bf)
        w2b_ref[...] = w2_ref[...].astype(bf)
        w3b_ref[...] = w3t_ref[...].T.astype(bf)

    @pl.when(g >= _NB)
    def _():
        t = g - _NB
        h = jnp.maximum(p_ref[pl.ds(t * _BT, _BT), :] + c0_ref[0:1, :], 0)
        h = jnp.maximum(jnp.dot(h, w1b_ref[...],
                                preferred_element_type=jnp.float32).astype(bf)
                        + b1_ref[...].astype(bf), 0)
        h = jnp.maximum(jnp.dot(h, w2b_ref[...],
                                preferred_element_type=jnp.float32).astype(bf)
                        + b2_ref[...].astype(bf), 0)
        o = jnp.dot(h, w3b_ref[...], preferred_element_type=jnp.float32)
        out_ref[...] = (o + b3_ref[...])[:, 0]


def kernel(numerical_features, categorical_features, emb_tables,
           W0, b0, W1, b1, W2, b2, W3, b3):
    del categorical_features
    n_num = numerical_features.shape[1]

    const = lambda i: (0, 0)
    const3 = lambda i: (0, 0, 0)
    out = pl.pallas_call(
        _mlp_kernel,
        grid=(2 * _NB,),
        in_specs=[
            pl.BlockSpec((n_num, _BT), lambda i: (0, jnp.minimum(i, _NB - 1))),
            pl.BlockSpec(emb_tables.shape, const3),
            pl.BlockSpec((16, 1024), const),
            pl.BlockSpec((1, b0.shape[0]), const),
            pl.BlockSpec(W1.shape, const),
            pl.BlockSpec((1, b1.shape[0]), const),
            pl.BlockSpec(W2.shape, const),
            pl.BlockSpec((1, b2.shape[0]), const),
            pl.BlockSpec((1, W3.shape[0]), const),
            pl.BlockSpec((1, 1), const),
            pl.BlockSpec(memory_space=pltpu.MemorySpace.HBM),
        ],
        out_specs=pl.BlockSpec((_BT,), lambda i: (jnp.maximum(i - _NB, 0),)),
        out_shape=jax.ShapeDtypeStruct((_B,), jnp.float32),
        scratch_shapes=[
            pltpu.VMEM((_B, b0.shape[0]), jnp.bfloat16),
            pltpu.VMEM((1, b0.shape[0]), jnp.bfloat16),
            pltpu.VMEM(W1.shape, jnp.bfloat16),
            pltpu.VMEM(W2.shape, jnp.bfloat16),
            pltpu.VMEM((W3.shape[0], 1), jnp.bfloat16),
            pltpu.VMEM(W0.shape, jnp.float32),
            pltpu.SemaphoreType.DMA,
        ],
    )(numerical_features.T, emb_tables, W0, b0.reshape(1, -1),
      W1, b1.reshape(1, -1), W2, b2.reshape(1, -1), W3.T, b3.reshape(1, 1),
      W0)
    return out


# speedup vs baseline: 1.5895x; 1.0253x over previous
import jax
import jax.numpy as jnp
from jax import lax
from jax.experimental import pallas as pl
from jax.experimental.pallas import tpu as pltpu

_B = 4096
_BT = 4096
_NB = _B // _BT
_W0_ROWS = 3341
_CHUNKS = [(0, 840), (840, 840), (1680, 840), (2520, 821)]


def _w0_copies(w0_any, w0_v, sem):
    return [pltpu.make_async_copy(w0_any.at[pl.ds(s, n), :],
                                  w0_v.at[pl.ds(s, n), :], sem)
            for s, n in _CHUNKS]


def _mlp_kernel(numt_ref, emb_ref, w0t16_ref, b0_ref,
                w1_ref, b1_ref, w2_ref, b2_ref, w3t_ref, b3_ref,
                w0_any,
                out_ref,
                p_ref, c0_ref, w1b_ref, w2b_ref, w3b_ref, w0_v, sem):
    bf = jnp.bfloat16
    n_num = numt_ref.shape[0]
    n_tab = emb_ref.shape[0]
    g = pl.program_id(0)

    @pl.when(g == 0)
    def _():
        for c in _w0_copies(w0_any, w0_v, sem):
            c.start()

    @pl.when(g < _NB)
    def _():
        xt = numt_ref[...].astype(bf)
        w0t = w0t16_ref[0:n_num, :].astype(bf)
        p = lax.dot_general(xt, w0t, (((0,), (0,)), ((), ())),
                            preferred_element_type=jnp.float32)
        p_ref[pl.ds(g * _BT, _BT), :] = p.astype(bf)

    @pl.when(g == _NB)
    def _():
        for c in _w0_copies(w0_any, w0_v, sem):
            c.wait()
        parts = [jnp.zeros((1, n_num), jnp.float32)]
        parts += [emb_ref[i, :, :] for i in range(n_tab)]
        emb_pad = jnp.concatenate(parts, axis=1)
        c0 = jnp.dot(emb_pad, w0_v[...], preferred_element_type=jnp.float32)
        c0_ref[...] = (c0 + b0_ref[...]).astype(bf)
        w1b_ref[...] = w1_ref[...].astype(bf)
        w2b_ref[...] = w2_ref[...].astype(bf)
        w3b_ref[...] = w3t_ref[...].T.astype(bf)

    @pl.when(g >= _NB)
    def _():
        t = g - _NB
        h = jnp.maximum(p_ref[pl.ds(t * _BT, _BT), :] + c0_ref[0:1, :], 0)
        h = jnp.maximum(jnp.dot(h, w1b_ref[...],
                                preferred_element_type=jnp.float32).astype(bf)
                        + b1_ref[...].astype(bf), 0)
        h = jnp.maximum(jnp.dot(h, w2b_ref[...],
                                preferred_element_type=jnp.float32).astype(bf)
                        + b2_ref[...].astype(bf), 0)
        o = jnp.dot(h, w3b_ref[...], preferred_element_type=jnp.float32)
        out_ref[...] = (o + b3_ref[...])[:, 0]


def kernel(numerical_features, categorical_features, emb_tables,
           W0, b0, W1, b1, W2, b2, W3, b3):
    del categorical_features
    n_num = numerical_features.shape[1]

    const = lambda i: (0, 0)
    const3 = lambda i: (0, 0, 0)
    out = pl.pallas_call(
        _mlp_kernel,
        grid=(2 * _NB,),
        in_specs=[
            pl.BlockSpec((n_num, _BT), lambda i: (0, jnp.minimum(i, _NB - 1))),
            pl.BlockSpec(emb_tables.shape, const3),
            pl.BlockSpec((16, 1024), const),
            pl.BlockSpec((1, b0.shape[0]), const),
            pl.BlockSpec(W1.shape, const),
            pl.BlockSpec((1, b1.shape[0]), const),
            pl.BlockSpec(W2.shape, const),
            pl.BlockSpec((1, b2.shape[0]), const),
            pl.BlockSpec((1, W3.shape[0]), const),
            pl.BlockSpec((1, 1), const),
            pl.BlockSpec(memory_space=pltpu.MemorySpace.HBM),
        ],
        out_specs=pl.BlockSpec((_BT,), lambda i: (jnp.maximum(i - _NB, 0),)),
        out_shape=jax.ShapeDtypeStruct((_B,), jnp.float32),
        scratch_shapes=[
            pltpu.VMEM((_B, b0.shape[0]), jnp.bfloat16),
            pltpu.VMEM((1, b0.shape[0]), jnp.bfloat16),
            pltpu.VMEM(W1.shape, jnp.bfloat16),
            pltpu.VMEM(W2.shape, jnp.bfloat16),
            pltpu.VMEM((W3.shape[0], 1), jnp.bfloat16),
            pltpu.VMEM(W0.shape, jnp.float32),
            pltpu.SemaphoreType.DMA,
        ],
    )(numerical_features.T, emb_tables, W0, b0.reshape(1, -1),
      W1, b1.reshape(1, -1), W2, b2.reshape(1, -1), W3.T, b3.reshape(1, 1),
      W0)
    return out


# speedup vs baseline: 1.7584x; 1.1062x over previous
import jax
import jax.numpy as jnp
from jax import lax
from jax.experimental import pallas as pl
from jax.experimental.pallas import tpu as pltpu

_B = 4096
_CHUNKS = [(0, 840), (840, 840), (1680, 840), (2520, 821)]

_CT = (((1,), (0,)), ((), ()))
_CTT = (((0,), (0,)), ((), ()))


def _w0_copies(w0_any, w0_v, sem):
    return [pltpu.make_async_copy(w0_any.at[pl.ds(s, n), :],
                                  w0_v.at[pl.ds(s, n), :], sem)
            for s, n in _CHUNKS]


def _mlp_kernel(numt_ref, emb_ref, w0t16_ref, b0_ref,
                w1_ref, b1_ref, w2_ref, b2_ref, w3t_ref, b3_ref,
                w0_any,
                out_ref,
                p_ref, c0_ref, w1b_ref, w2b_ref, w3b_ref, w0_v, sem):
    bf = jnp.bfloat16
    n_num = numt_ref.shape[0]
    n_tab = emb_ref.shape[0]
    g = pl.program_id(0)

    @pl.when(g == 0)
    def _():
        for c in _w0_copies(w0_any, w0_v, sem):
            c.start()
        xt = numt_ref[...].astype(bf)
        w0t = w0t16_ref[0:n_num, :].astype(bf)
        p = lax.dot_general(w0t, xt, _CTT,
                            preferred_element_type=jnp.float32)
        p_ref[...] = p.astype(bf)

    @pl.when(g == 1)
    def _():
        for c in _w0_copies(w0_any, w0_v, sem):
            c.wait()
        parts = [jnp.zeros((1, n_num), jnp.float32)]
        parts += [emb_ref[i, :, :] for i in range(n_tab)]
        emb_pad = jnp.concatenate(parts, axis=1)
        c0 = jnp.dot(emb_pad, w0_v[...],
                     preferred_element_type=jnp.float32)
        c0_ref[...] = (c0 + b0_ref[...]).T.astype(bf)
        w1b_ref[...] = w1_ref[...].astype(bf)
        w2b_ref[...] = w2_ref[...].astype(bf)
        w3b_ref[...] = w3t_ref[...].astype(bf)
        b1c = b1_ref[...].T.astype(bf)
        b2c = b2_ref[...].T.astype(bf)

        h = jnp.maximum(p_ref[...] + c0_ref[...], 0)
        h = jnp.maximum(
            lax.dot_general(w1b_ref[...], h, _CTT,
                            preferred_element_type=jnp.float32).astype(bf)
            + b1c, 0)
        h = jnp.maximum(
            lax.dot_general(w2b_ref[...], h, _CTT,
                            preferred_element_type=jnp.float32).astype(bf)
            + b2c, 0)
        o = lax.dot_general(w3b_ref[...], h, _CT,
                            preferred_element_type=jnp.float32)
        out_ref[...] = (o + b3_ref[...])[0, :]


def kernel(numerical_features, categorical_features, emb_tables,
           W0, b0, W1, b1, W2, b2, W3, b3):
    del categorical_features
    n_num = numerical_features.shape[1]

    const = lambda i: (0, 0)
    const3 = lambda i: (0, 0, 0)
    out = pl.pallas_call(
        _mlp_kernel,
        grid=(2,),
        in_specs=[
            pl.BlockSpec((n_num, _B), const),
            pl.BlockSpec(emb_tables.shape, const3),
            pl.BlockSpec((16, 1024), const),
            pl.BlockSpec((1, b0.shape[0]), const),
            pl.BlockSpec(W1.shape, const),
            pl.BlockSpec((1, b1.shape[0]), const),
            pl.BlockSpec(W2.shape, const),
            pl.BlockSpec((1, b2.shape[0]), const),
            pl.BlockSpec((1, W3.shape[0]), const),
            pl.BlockSpec((1, 1), const),
            pl.BlockSpec(memory_space=pltpu.MemorySpace.HBM),
        ],
        out_specs=pl.BlockSpec((_B,), lambda i: (0,)),
        out_shape=jax.ShapeDtypeStruct((_B,), jnp.float32),
        scratch_shapes=[
            pltpu.VMEM((b0.shape[0], _B), jnp.bfloat16),
            pltpu.VMEM((b0.shape[0], 1), jnp.bfloat16),
            pltpu.VMEM(W1.shape, jnp.bfloat16),
            pltpu.VMEM(W2.shape, jnp.bfloat16),
            pltpu.VMEM((1, W3.shape[0]), jnp.bfloat16),
            pltpu.VMEM(W0.shape, jnp.float32),
            pltpu.SemaphoreType.DMA,
        ],
    )(numerical_features.T, emb_tables, W0, b0.reshape(1, -1),
      W1, b1.reshape(1, -1), W2, b2.reshape(1, -1), W3.T, b3.reshape(1, 1),
      W0)
    return out
